# baseline (speedup 1.0000x reference)
import jax
import jax.numpy as jnp
from jax.experimental import pallas as pl

_SEQ = 19
_N_ROUTED = 8
_GRID = 4
_PAT = _SEQ * _N_ROUTED


def _expert_of(pos):
    return jnp.where(
        (pos == 0) | (pos == _SEQ - 1), 0, jnp.where(pos <= 10, 1, 2)
    )


def _gate_body(w_ref, i_ref, p_ref):
    e = (
        jax.lax.broadcasted_iota(jnp.int32, (_PAT, 128), 0) * 128
        + jax.lax.broadcasted_iota(jnp.int32, (_PAT, 128), 1)
    )
    idx_tile = _expert_of(e % _SEQ)
    i_ref[...] = jnp.concatenate([idx_tile] * (i_ref.shape[0] // _PAT), axis=0)

    w_ref[...] = jnp.ones(w_ref.shape, jnp.float32)

    n_lanes = _N_ROUTED // _GRID
    rows_per_lane = p_ref.shape[0] // n_lanes
    t = (
        jax.lax.broadcasted_iota(jnp.int32, (_PAT, 128), 0) * 128
        + jax.lax.broadcasted_iota(jnp.int32, (_PAT, 128), 1)
    )
    exp_tile = _expert_of(t % _SEQ)
    q0 = pl.program_id(0) * n_lanes
    chunks = []
    for j in range(n_lanes):
        one = (exp_tile == (q0 + j)).astype(jnp.float32)
        chunks.extend([one] * (rows_per_lane // _PAT))
    p_ref[...] = jnp.concatenate(chunks, axis=0)


def kernel(x):
    n = x.shape[0]
    iw_rows = n // 128 // _GRID
    p_rows = n * _N_ROUTED // 128 // _GRID
    weights, indices, probs_flat = pl.pallas_call(
        _gate_body,
        grid=(_GRID,),
        out_specs=[
            pl.BlockSpec((iw_rows, 128), lambda i: (i, 0)),
            pl.BlockSpec((iw_rows, 128), lambda i: (i, 0)),
            pl.BlockSpec((p_rows, 128), lambda i: (i, 0)),
        ],
        out_shape=[
            jax.ShapeDtypeStruct((n // 128, 128), jnp.float32),
            jax.ShapeDtypeStruct((n // 128, 128), jnp.int32),
            jax.ShapeDtypeStruct((n * _N_ROUTED // 128, 128), jnp.float32),
        ],
    )()
    return (
        weights.reshape(n, 1),
        indices.reshape(n, 1),
        probs_flat.reshape(_N_ROUTED, n).T,
    )


# speedup vs baseline: 2.6114x; 2.6114x over previous
import jax
import jax.numpy as jnp
from jax.experimental import pallas as pl
from jax.experimental.pallas import tpu as pltpu

_SEQ = 19
_N_ROUTED = 8
_GRID = 8
_PAT = _SEQ * _N_ROUTED


def _expert_of(pos):
    return jnp.where(
        (pos == 0) | (pos == _SEQ - 1), 0, jnp.where(pos <= 10, 1, 2)
    )


def _gate_body(w_ref, i_ref, p_ref):
    e = (
        jax.lax.broadcasted_iota(jnp.int32, (_PAT, 128), 0) * 128
        + jax.lax.broadcasted_iota(jnp.int32, (_PAT, 128), 1)
    )
    idx_tile = _expert_of(e % _SEQ)
    i_ref[...] = jnp.concatenate([idx_tile] * (i_ref.shape[0] // _PAT), axis=0)

    w_ref[...] = jnp.ones(w_ref.shape, jnp.float32)

    pos = jax.lax.broadcasted_iota(jnp.int32, (8, _SEQ * 128), 1) % _SEQ
    lane = jax.lax.broadcasted_iota(jnp.int32, (8, _SEQ * 128), 0)
    prob_tile = (lane == _expert_of(pos)).astype(jnp.float32)
    p_ref[...] = jnp.concatenate(
        [prob_tile] * (p_ref.shape[1] // (_SEQ * 128)), axis=1
    )


def kernel(x):
    n = x.shape[0]
    iw_rows = n // 128 // _GRID
    p_cols = n // _GRID
    weights, indices, probs_t = pl.pallas_call(
        _gate_body,
        grid=(_GRID,),
        compiler_params=pltpu.CompilerParams(dimension_semantics=("parallel",)),
        out_specs=[
            pl.BlockSpec((iw_rows, 128), lambda i: (i, 0)),
            pl.BlockSpec((iw_rows, 128), lambda i: (i, 0)),
            pl.BlockSpec((_N_ROUTED, p_cols), lambda i: (0, i)),
        ],
        out_shape=[
            jax.ShapeDtypeStruct((n // 128, 128), jnp.float32),
            jax.ShapeDtypeStruct((n // 128, 128), jnp.int32),
            jax.ShapeDtypeStruct((_N_ROUTED, n), jnp.float32),
        ],
    )()
    return (
        weights.reshape(n, 1),
        indices.reshape(n, 1),
        probs_t.T,
    )


# speedup vs baseline: 3.1283x; 1.1980x over previous
import jax
import jax.numpy as jnp
from jax.experimental import pallas as pl

_SEQ = 19
_N_ROUTED = 8
_GRID = 4
_PAT = _SEQ * _N_ROUTED


def _expert_of(pos):
    return jnp.where(
        (pos == 0) | (pos == _SEQ - 1), 0, jnp.where(pos <= 10, 1, 2)
    )


def _gate_body(w_ref, i_ref, p_ref):
    e = (
        jax.lax.broadcasted_iota(jnp.int32, (_PAT, 128), 0) * 128
        + jax.lax.broadcasted_iota(jnp.int32, (_PAT, 128), 1)
    )
    idx_tile = _expert_of(e % _SEQ)
    i_ref[...] = jnp.concatenate([idx_tile] * (i_ref.shape[0] // _PAT), axis=0)

    w_ref[...] = jnp.ones(w_ref.shape, jnp.float32)

    pos = jax.lax.broadcasted_iota(jnp.int32, (8, _SEQ * 128), 1) % _SEQ
    lane = jax.lax.broadcasted_iota(jnp.int32, (8, _SEQ * 128), 0)
    prob_tile = (lane == _expert_of(pos)).astype(jnp.float32)
    p_ref[...] = jnp.concatenate(
        [prob_tile] * (p_ref.shape[1] // (_SEQ * 128)), axis=1
    )


def kernel(x):
    n = x.shape[0]
    iw_rows = n // 128 // _GRID
    p_cols = n // _GRID
    weights, indices, probs_t = pl.pallas_call(
        _gate_body,
        grid=(_GRID,),
        out_specs=[
            pl.BlockSpec((iw_rows, 128), lambda i: (i, 0)),
            pl.BlockSpec((iw_rows, 128), lambda i: (i, 0)),
            pl.BlockSpec((_N_ROUTED, p_cols), lambda i: (0, i)),
        ],
        out_shape=[
            jax.ShapeDtypeStruct((n // 128, 128), jnp.float32),
            jax.ShapeDtypeStruct((n // 128, 128), jnp.int32),
            jax.ShapeDtypeStruct((_N_ROUTED, n), jnp.float32),
        ],
    )()
    return (
        weights.reshape(n, 1),
        indices.reshape(n, 1),
        probs_t.T,
    )
